# baseline (speedup 1.0000x reference)
import functools

import jax
import jax.numpy as jnp
from jax import lax
from jax.experimental import pallas as pl
from jax.experimental.pallas import tpu as pltpu
from jax.experimental.pallas import tpu_sc as plsc

DIM = 32
B = 4096
S = 200
NW = 32
BW = B // NW
PER_W = BW * S


@functools.cache
def _make_gather(num_embeddings: int):
    mesh = plsc.VectorSubcoreMesh(core_axis_name="c", subcore_axis_name="s")

    @functools.partial(
        pl.kernel,
        mesh=mesh,
        out_type=jax.ShapeDtypeStruct((S, 4, NW, 8, BW), jnp.float32),
        scratch_types=[
            pltpu.VMEM((PER_W,), jnp.int32),
            pltpu.VMEM((2, BW), jnp.int32),
            pltpu.VMEM((2, BW, DIM), jnp.float32),
            pltpu.VMEM((2, 4, 8, BW), jnp.float32),
            pltpu.SemaphoreType.DMA,
            pltpu.SemaphoreType.DMA,
        ],
        compiler_params=pltpu.CompilerParams(
            use_tc_tiling_on_sc=False, needs_layout_passes=False),
    )
    def gather_kernel(table_hbm, idx_hbm, out_hbm, idx_v, idxcol_v, rows_v,
                      trows_v, gsem, ssem):
        wid = lax.axis_index("s") * 2 + lax.axis_index("c")

        pltpu.sync_copy(idx_hbm.at[pl.ds(wid * PER_W, PER_W)], idx_v)

        lane = lax.iota(jnp.int32, 16)
        repack_base = [lane * S + (16 * j * S) for j in range(8)]
        trans_rows = [lane + 16 * j for j in range(8)]

        def repack(s, p):
            for j in range(8):
                v = plsc.load_gather(idx_v, [repack_base[j] + s])
                idxcol_v[p, pl.ds(16 * j, 16)] = v

        def start_gather(p):
            return pltpu.async_copy(
                table_hbm.at[idxcol_v.at[p]], rows_v.at[p], gsem)

        def transpose(p):
            for d in range(DIM):
                col = jnp.full((16,), d, jnp.int32)
                for j in range(8):
                    v = plsc.load_gather(rows_v.at[p], [trans_rows[j], col])
                    trows_v[p, d // 8, d % 8, pl.ds(16 * j, 16)] = v

        def start_store(s, p):
            return pltpu.async_copy(
                trows_v.at[p], out_hbm.at[s, :, wid], ssem)

        def wait_gather(p):
            pltpu.make_async_copy(
                table_hbm.at[idxcol_v.at[p]], rows_v.at[p], gsem).wait()

        def wait_store(s, p):
            pltpu.make_async_copy(
                trows_v.at[p], out_hbm.at[s, :, wid], ssem).wait()

        repack(0, 0)
        start_gather(0)

        def body(g, carry):
            for phase in range(2):
                s = 2 * g + phase
                p = phase
                q = 1 - phase
                nxt = s + 1

                @pl.when(nxt < S)
                def _():
                    repack(nxt, q)
                    start_gather(q)

                wait_gather(p)

                @pl.when(s >= 2)
                def _():
                    wait_store(s, p)

                transpose(p)
                start_store(s, p)
            return carry

        lax.fori_loop(0, S // 2, body, 0)
        wait_store(0, 0)
        wait_store(0, 1)

    return gather_kernel


def kernel(input, weight):
    idx = input.reshape(-1).astype(jnp.int32)
    out5d = _make_gather(weight.shape[0])(weight, idx)
    return out5d.transpose(2, 4, 0, 1, 3).reshape(B, S, DIM)


# speedup vs baseline: 1.0024x; 1.0024x over previous
import functools

import jax
import jax.numpy as jnp
from jax import lax
from jax.experimental import pallas as pl
from jax.experimental.pallas import tpu as pltpu
from jax.experimental.pallas import tpu_sc as plsc

DIM = 32
B = 4096
S = 200
NW = 32
BW = B // NW
G = 10
NG = S // G
GI = G * BW


@functools.cache
def _make_gather(num_embeddings: int):
    mesh = plsc.VectorSubcoreMesh(core_axis_name="c", subcore_axis_name="s")

    @functools.partial(
        pl.kernel,
        mesh=mesh,
        out_type=jax.ShapeDtypeStruct((S, 4, NW, 8, BW), jnp.float32),
        scratch_types=[
            pltpu.VMEM((2, GI), jnp.int32),
            pltpu.VMEM((2, GI, DIM), jnp.float32),
            pltpu.VMEM((G, 4, 8, BW), jnp.float32),
            pltpu.SemaphoreType.DMA,
            pltpu.SemaphoreType.DMA,
            pltpu.SemaphoreType.DMA,
        ],
        compiler_params=pltpu.CompilerParams(
            use_tc_tiling_on_sc=False, needs_layout_passes=False),
    )
    def gather_kernel(table_hbm, idx_hbm, out_hbm, idx_v, rows_v, trows_v,
                      isem, gsem, ssem):
        wid = lax.axis_index("s") * 2 + lax.axis_index("c")
        ibase = wid * (S * BW)

        lane = lax.iota(jnp.int32, 16)
        trans_rows = [lane + 16 * j for j in range(8)]

        def start_idx(g, p):
            return pltpu.async_copy(
                idx_hbm.at[pl.ds(ibase + g * GI, GI)], idx_v.at[p], isem)

        def wait_idx(p):
            pltpu.make_async_copy(
                idx_hbm.at[pl.ds(ibase, GI)], idx_v.at[p], isem).wait()

        def start_gather(p):
            return pltpu.async_copy(
                table_hbm.at[idx_v.at[p]], rows_v.at[p], gsem)

        def wait_gather(p):
            pltpu.make_async_copy(
                table_hbm.at[idx_v.at[p]], rows_v.at[p], gsem).wait()

        def start_store(g):
            return pltpu.async_copy(
                trows_v, out_hbm.at[pl.ds(g * G, G), :, wid], ssem)

        def wait_store():
            pltpu.make_async_copy(
                trows_v, out_hbm.at[pl.ds(0, G), :, wid], ssem).wait()

        def transpose(p):
            def unit(u, carry):
                base = u * BW
                rv = [trans_rows[j] + base for j in range(8)]
                for d in range(DIM):
                    col = jnp.full((16,), d, jnp.int32)
                    for j in range(8):
                        v = plsc.load_gather(rows_v.at[p], [rv[j], col])
                        trows_v[u, d // 8, d % 8, pl.ds(16 * j, 16)] = v
                return carry

            lax.fori_loop(0, G, unit, 0)

        start_idx(0, 0)
        wait_idx(0)
        start_gather(0)
        start_idx(1, 1)

        def body(h, carry):
            for phase in range(2):
                p = phase
                q = 1 - phase
                g = 2 * h + phase
                wait_gather(p)

                @pl.when(g + 1 < NG)
                def _():
                    wait_idx(q)
                    start_gather(q)

                @pl.when(g + 2 < NG)
                def _():
                    start_idx(g + 2, p)

                @pl.when(g >= 1)
                def _():
                    wait_store()

                transpose(p)
                start_store(g)
            return carry

        lax.fori_loop(0, NG // 2, body, 0)
        wait_store()

    return gather_kernel


def kernel(input, weight):
    idx_blk = (input.astype(jnp.int32)
               .reshape(NW, BW, S)
               .transpose(0, 2, 1)
               .reshape(-1))
    out5d = _make_gather(weight.shape[0])(weight, idx_blk)
    return out5d.transpose(2, 4, 0, 1, 3).reshape(B, S, DIM)


# speedup vs baseline: 1.1285x; 1.1258x over previous
import functools

import jax
import jax.numpy as jnp
from jax import lax
from jax.experimental import pallas as pl
from jax.experimental.pallas import tpu as pltpu
from jax.experimental.pallas import tpu_sc as plsc

DIM = 32
B = 4096
S = 200
NW = 32
BW = B // NW
G = 10
NG = S // G
GI = G * BW


@functools.cache
def _make_gather(num_embeddings: int):
    mesh = plsc.VectorSubcoreMesh(core_axis_name="c", subcore_axis_name="s")

    @functools.partial(
        pl.kernel,
        mesh=mesh,
        out_type=jax.ShapeDtypeStruct((S, 4, NW, 8, BW), jnp.float32),
        scratch_types=[
            pltpu.VMEM((2, GI), jnp.int32),
            pltpu.VMEM((2, GI, DIM), jnp.float32),
            pltpu.VMEM((G, 4, 8, BW), jnp.float32),
            pltpu.SemaphoreType.DMA,
            pltpu.SemaphoreType.DMA,
            pltpu.SemaphoreType.DMA,
        ],
        compiler_params=pltpu.CompilerParams(
            use_tc_tiling_on_sc=False, needs_layout_passes=False),
    )
    def gather_kernel(table_hbm, idx_hbm, out_hbm, idx_v, rows_v, trows_v,
                      isem, gsem, ssem):
        wid = lax.axis_index("s") * 2 + lax.axis_index("c")
        ibase = wid * (S * BW)

        lane = lax.iota(jnp.int32, 16)
        trans_rows = [lane + 16 * j for j in range(8)]

        def start_idx(g, p):
            return pltpu.async_copy(
                idx_hbm.at[pl.ds(ibase + g * GI, GI)], idx_v.at[p], isem)

        def wait_idx(p):
            pltpu.make_async_copy(
                idx_hbm.at[pl.ds(ibase, GI)], idx_v.at[p], isem).wait()

        def start_gather(p):
            return pltpu.async_copy(
                table_hbm.at[idx_v.at[p]], rows_v.at[p], gsem)

        def wait_gather(p):
            pltpu.make_async_copy(
                table_hbm.at[idx_v.at[p]], rows_v.at[p], gsem).wait()

        def start_store(g):
            return pltpu.async_copy(
                trows_v, out_hbm.at[pl.ds(g * G, G), :, wid], ssem)

        def wait_store():
            pltpu.make_async_copy(
                trows_v, out_hbm.at[pl.ds(0, G), :, wid], ssem).wait()

        def transpose(p):
            def unit(u, carry):
                base = u * BW
                rv = [trans_rows[j] + base for j in range(8)]
                for d in range(DIM):
                    col = jnp.full((16,), d, jnp.int32)
                    vs = [plsc.load_gather(rows_v.at[p], [rv[j], col])
                          for j in range(8)]
                    for j in range(8):
                        trows_v[u, d // 8, d % 8, pl.ds(16 * j, 16)] = vs[j]
                return carry

            lax.fori_loop(0, G, unit, 0)

        start_idx(0, 0)
        wait_idx(0)
        start_gather(0)
        start_idx(1, 1)

        def body(h, carry):
            for phase in range(2):
                p = phase
                q = 1 - phase
                g = 2 * h + phase
                wait_gather(p)

                @pl.when(g + 1 < NG)
                def _():
                    wait_idx(q)
                    start_gather(q)

                @pl.when(g + 2 < NG)
                def _():
                    start_idx(g + 2, p)

                @pl.when(g >= 1)
                def _():
                    wait_store()

                transpose(p)
                start_store(g)
            return carry

        lax.fori_loop(0, NG // 2, body, 0)
        wait_store()

    return gather_kernel


def kernel(input, weight):
    idx_blk = (input.astype(jnp.int32)
               .reshape(NW, BW, S)
               .transpose(0, 2, 1)
               .reshape(-1))
    out5d = _make_gather(weight.shape[0])(weight, idx_blk)
    return out5d.transpose(2, 4, 0, 1, 3).reshape(B, S, DIM)


# speedup vs baseline: 1.9294x; 1.7097x over previous
import functools

import jax
import jax.numpy as jnp
from jax import lax
from jax.experimental import pallas as pl
from jax.experimental.pallas import tpu as pltpu
from jax.experimental.pallas import tpu_sc as plsc

DIM = 32
B = 4096
S = 200
NW = 32
BW = B // NW
G = 10
NG = S // G
GI = G * BW


@functools.cache
def _make_gather(num_embeddings: int):
    mesh = plsc.VectorSubcoreMesh(core_axis_name="c", subcore_axis_name="s")

    @functools.partial(
        pl.kernel,
        mesh=mesh,
        out_type=jax.ShapeDtypeStruct((S, 4, NW, 8, BW), jnp.float32),
        scratch_types=[
            pltpu.VMEM((2, GI), jnp.int32),
            pltpu.VMEM((2, GI, DIM), jnp.float32),
            pltpu.VMEM((G, 4, 8, BW + 1), jnp.float32),
            pltpu.SemaphoreType.DMA,
            pltpu.SemaphoreType.DMA,
            pltpu.SemaphoreType.DMA,
        ],
        compiler_params=pltpu.CompilerParams(
            use_tc_tiling_on_sc=False, needs_layout_passes=False),
    )
    def gather_kernel(table_hbm, idx_hbm, out_hbm, idx_v, rows_v, trows_v,
                      isem, gsem, ssem):
        wid = lax.axis_index("s") * 2 + lax.axis_index("c")
        ibase = wid * (S * BW)

        lane = lax.iota(jnp.int32, 16)
        tr_vec = [(lane + 16 * h) // 8 for h in range(2)]
        r_vec = [(lane + 16 * h) % 8 for h in range(2)]

        def start_idx(g, p):
            return pltpu.async_copy(
                idx_hbm.at[pl.ds(ibase + g * GI, GI)], idx_v.at[p], isem)

        def wait_idx(p):
            pltpu.make_async_copy(
                idx_hbm.at[pl.ds(ibase, GI)], idx_v.at[p], isem).wait()

        def start_gather(p):
            return pltpu.async_copy(
                table_hbm.at[idx_v.at[p]], rows_v.at[p], gsem)

        def wait_gather(p):
            pltpu.make_async_copy(
                table_hbm.at[idx_v.at[p]], rows_v.at[p], gsem).wait()

        def start_store(g):
            return pltpu.async_copy(
                trows_v.at[:, :, :, pl.ds(0, BW)],
                out_hbm.at[pl.ds(g * G, G), :, wid], ssem)

        def wait_store():
            pltpu.make_async_copy(
                trows_v.at[:, :, :, pl.ds(0, BW)],
                out_hbm.at[pl.ds(0, G), :, wid], ssem).wait()

        def transpose(p):
            def unit(u, carry):
                base = u * BW
                uvec = jnp.full((16,), 0, jnp.int32) + u

                def colblk(cb, carry2):
                    c0 = cb * 8
                    vs = [[rows_v[p, base + c0 + cc, pl.ds(16 * h, 16)]
                           for h in range(2)] for cc in range(8)]
                    for cc in range(8):
                        cvec = jnp.full((16,), 0, jnp.int32) + (c0 + cc)
                        for h in range(2):
                            plsc.store_scatter(
                                trows_v, [uvec, tr_vec[h], r_vec[h], cvec],
                                vs[cc][h])
                    return carry2

                return lax.fori_loop(0, BW // 8, colblk, carry)

            lax.fori_loop(0, G, unit, 0)

        start_idx(0, 0)
        wait_idx(0)
        start_gather(0)
        start_idx(1, 1)

        def body(h, carry):
            for phase in range(2):
                p = phase
                q = 1 - phase
                g = 2 * h + phase
                wait_gather(p)

                @pl.when(g + 1 < NG)
                def _():
                    wait_idx(q)
                    start_gather(q)

                @pl.when(g + 2 < NG)
                def _():
                    start_idx(g + 2, p)

                @pl.when(g >= 1)
                def _():
                    wait_store()

                transpose(p)
                start_store(g)
            return carry

        lax.fori_loop(0, NG // 2, body, 0)
        wait_store()

    return gather_kernel


def kernel(input, weight):
    idx_blk = (input.astype(jnp.int32)
               .reshape(NW, BW, S)
               .transpose(0, 2, 1)
               .reshape(-1))
    out5d = _make_gather(weight.shape[0])(weight, idx_blk)
    return out5d.transpose(2, 4, 0, 1, 3).reshape(B, S, DIM)
